# Initial kernel scaffold; baseline (speedup 1.0000x reference)
import jax
import jax.numpy as jnp
from jax.experimental import pallas as pl


def kernel(x, edge_index, W1, b1, W2, b2):
    raise NotImplementedError("write your pallas kernel here")


# speedup vs baseline: 25.3929x; 25.3929x over previous
import functools

import jax
import jax.numpy as jnp
from jax import lax
from jax.experimental import pallas as pl
from jax.experimental.pallas import tpu as pltpu
from jax.experimental.pallas import tpu_sc as plsc

N = 10000
NE = 320000
D_IN = 128
D_HID = 128
D_OUT = 40

NW = 32
EPT = NE // NW
K = 80
NCH = EPT // K

RPW = 640
TAIL = N - 15 * RPW

_mesh = plsc.VectorSubcoreMesh(core_axis_name="c", subcore_axis_name="s")


def _zero_acc(zeros_hbm, acc, s):
    @pl.when(s < 15)
    def _():
        pltpu.sync_copy(zeros_hbm, acc.at[pl.ds(s * RPW, RPW)])

    @pl.when(s == 15)
    def _():
        pltpu.sync_copy(zeros_hbm.at[pl.ds(0, TAIL)], acc.at[pl.ds(15 * RPW, TAIL)])


def _write_out(acc, out_hbm, c, s):
    @pl.when(s < 15)
    def _():
        pltpu.sync_copy(acc.at[pl.ds(s * RPW, RPW)], out_hbm.at[c, pl.ds(s * RPW, RPW)])

    @pl.when(s == 15)
    def _():
        pltpu.sync_copy(acc.at[pl.ds(15 * RPW, TAIL)], out_hbm.at[c, pl.ds(15 * RPW, TAIL)])


DEGW = 16


def _make_deg_kernel():
    @functools.partial(
        pl.kernel,
        out_type=jax.ShapeDtypeStruct((2, N, DEGW), jnp.float32),
        mesh=_mesh,
        scratch_types=[
            pltpu.VMEM((NCH, K), jnp.int32),
            pltpu.VMEM((K, DEGW), jnp.float32),
            pltpu.VMEM_SHARED((N, DEGW), jnp.float32),
        ],
        compiler_params=pltpu.CompilerParams(use_tc_tiling_on_sc=False),
    )
    def deg_kernel(dsts_hbm, ones_hbm, zeros_hbm, out_hbm, dst_v, ones_v, acc):
        c = lax.axis_index("c")
        s = lax.axis_index("s")
        w = c * 16 + s
        pltpu.sync_copy(dsts_hbm.at[w], dst_v)
        pltpu.sync_copy(ones_hbm, ones_v)
        _zero_acc(zeros_hbm, acc, s)
        plsc.subcore_barrier()

        def body(j, carry):
            pltpu.sync_copy(ones_v, acc.at[dst_v.at[j]], add=True)
            return carry

        lax.fori_loop(0, NCH, body, 0)
        plsc.subcore_barrier()
        _write_out(acc, out_hbm, c, s)

    return deg_kernel


def _make_agg_kernel(D):
    @functools.partial(
        pl.kernel,
        out_type=jax.ShapeDtypeStruct((2, N, D), jnp.float32),
        mesh=_mesh,
        scratch_types=[
            pltpu.VMEM((2, K), jnp.int32),
            pltpu.VMEM((2, K), jnp.int32),
            pltpu.VMEM((2, K, D), jnp.float32),
            pltpu.VMEM_SHARED((N, D), jnp.float32),
            pltpu.SemaphoreType.DMA,
            pltpu.SemaphoreType.DMA,
        ],
        compiler_params=pltpu.CompilerParams(use_tc_tiling_on_sc=False),
    )
    def agg_kernel(hs_hbm, srcs_hbm, dsts_hbm, zeros_hbm, out_hbm,
                   src_v, dst_v, stage, acc, gsem, isem):
        c = lax.axis_index("c")
        s = lax.axis_index("s")
        w = c * 16 + s
        _zero_acc(zeros_hbm, acc, s)

        pltpu.sync_copy(srcs_hbm.at[w, 0], src_v.at[0])
        pltpu.sync_copy(dsts_hbm.at[w, 0], dst_v.at[0])
        pltpu.async_copy(srcs_hbm.at[w, 1], src_v.at[1], isem)
        pltpu.async_copy(dsts_hbm.at[w, 1], dst_v.at[1], isem)
        plsc.subcore_barrier()
        pltpu.async_copy(hs_hbm.at[src_v.at[0]], stage.at[0], gsem)

        def wait_gather(p):
            pltpu.make_async_copy(hs_hbm.at[src_v.at[p]], stage.at[p], gsem).wait()

        def wait_idx(p):
            pltpu.make_async_copy(srcs_hbm.at[w, 0], src_v.at[p], isem).wait()
            pltpu.make_async_copy(dsts_hbm.at[w, 0], dst_v.at[p], isem).wait()

        def scatter(p):
            pltpu.sync_copy(stage.at[p], acc.at[dst_v.at[p]], add=True)

        def body(j, carry):
            p = lax.rem(j, 2)
            q = 1 - p
            wait_gather(p)
            wait_idx(q)
            pltpu.async_copy(hs_hbm.at[src_v.at[q]], stage.at[q], gsem)
            scatter(p)
            pltpu.async_copy(srcs_hbm.at[w, j + 2], src_v.at[p], isem)
            pltpu.async_copy(dsts_hbm.at[w, j + 2], dst_v.at[p], isem)
            return carry

        lax.fori_loop(0, NCH - 2, body, 0)

        p = (NCH - 2) % 2
        q = 1 - p
        wait_gather(p)
        wait_idx(q)
        pltpu.async_copy(hs_hbm.at[src_v.at[q]], stage.at[q], gsem)
        scatter(p)
        wait_gather(q)
        scatter(q)

        plsc.subcore_barrier()
        _write_out(acc, out_hbm, c, s)

    return agg_kernel


_deg_kernel = _make_deg_kernel()
_agg128 = _make_agg_kernel(D_HID)
_agg40 = _make_agg_kernel(D_OUT)

_TCB = 1000


def _tc1_body(deg_ref, x_ref, w_ref, hs_ref, dinv_ref):
    deg = deg_ref[0, :, 0:1] + deg_ref[1, :, 0:1] + 1.0
    dinv = lax.rsqrt(deg)
    h = jnp.dot(x_ref[...], w_ref[...], preferred_element_type=jnp.float32)
    hs_ref[...] = h * dinv
    dinv_ref[...] = dinv


def _tc1(degp, x, W1):
    grid = (N // _TCB,)
    return pl.pallas_call(
        _tc1_body,
        grid=grid,
        in_specs=[
            pl.BlockSpec((2, _TCB, DEGW), lambda i: (0, i, 0)),
            pl.BlockSpec((_TCB, D_IN), lambda i: (i, 0)),
            pl.BlockSpec((D_IN, D_HID), lambda i: (0, 0)),
        ],
        out_specs=[
            pl.BlockSpec((_TCB, D_HID), lambda i: (i, 0)),
            pl.BlockSpec((_TCB, 1), lambda i: (i, 0)),
        ],
        out_shape=[
            jax.ShapeDtypeStruct((N, D_HID), jnp.float32),
            jax.ShapeDtypeStruct((N, 1), jnp.float32),
        ],
    )(degp, x, W1)


def _tc2_body(agg_ref, hs1_ref, dinv_ref, b1_ref, w2_ref, hs2_ref):
    dinv = dinv_ref[...]
    o = (agg_ref[0] + agg_ref[1] + hs1_ref[...]) * dinv + b1_ref[...]
    o = jnp.maximum(o, 0.0)
    h2 = jnp.dot(o, w2_ref[...], preferred_element_type=jnp.float32)
    hs2_ref[...] = h2 * dinv


def _tc2(agg, hs1, dinv, b1, W2):
    grid = (N // _TCB,)
    return pl.pallas_call(
        _tc2_body,
        grid=grid,
        in_specs=[
            pl.BlockSpec((2, _TCB, D_HID), lambda i: (0, i, 0)),
            pl.BlockSpec((_TCB, D_HID), lambda i: (i, 0)),
            pl.BlockSpec((_TCB, 1), lambda i: (i, 0)),
            pl.BlockSpec((1, D_HID), lambda i: (0, 0)),
            pl.BlockSpec((D_HID, D_OUT), lambda i: (0, 0)),
        ],
        out_specs=pl.BlockSpec((_TCB, D_OUT), lambda i: (i, 0)),
        out_shape=jax.ShapeDtypeStruct((N, D_OUT), jnp.float32),
    )(agg, hs1, dinv, b1, W2)


def _tc3_body(agg_ref, hs2_ref, dinv_ref, b2_ref, out_ref):
    z = (agg_ref[0] + agg_ref[1] + hs2_ref[...]) * dinv_ref[...] + b2_ref[...]
    m = jnp.max(z, axis=1, keepdims=True)
    e = jnp.exp(z - m)
    lse = jnp.log(jnp.sum(e, axis=1, keepdims=True)) + m
    out_ref[...] = z - lse


def _tc3(agg, hs2, dinv, b2):
    grid = (N // _TCB,)
    return pl.pallas_call(
        _tc3_body,
        grid=grid,
        in_specs=[
            pl.BlockSpec((2, _TCB, D_OUT), lambda i: (0, i, 0)),
            pl.BlockSpec((_TCB, D_OUT), lambda i: (i, 0)),
            pl.BlockSpec((_TCB, 1), lambda i: (i, 0)),
            pl.BlockSpec((1, D_OUT), lambda i: (0, 0)),
        ],
        out_specs=pl.BlockSpec((_TCB, D_OUT), lambda i: (i, 0)),
        out_shape=jax.ShapeDtypeStruct((N, D_OUT), jnp.float32),
    )(agg, hs2, dinv, b2)


def kernel(x, edge_index, W1, b1, W2, b2):
    e = edge_index.astype(jnp.int32)
    src_r = e[0].reshape(NW, NCH, K)
    dst_r = e[1].reshape(NW, NCH, K)

    ones_c = jnp.ones((K, DEGW), jnp.float32)
    zeros_c = jnp.zeros((RPW, DEGW), jnp.float32)
    zeros_h = jnp.zeros((RPW, D_HID), jnp.float32)
    zeros_o = jnp.zeros((RPW, D_OUT), jnp.float32)

    degp = _deg_kernel(dst_r, ones_c, zeros_c)
    hs1, dinv = _tc1(degp, x, W1)
    agg1 = _agg128(hs1, src_r, dst_r, zeros_h)
    hs2 = _tc2(agg1, hs1, dinv, b1.reshape(1, D_HID), W2)
    agg2 = _agg40(hs2, src_r, dst_r, zeros_o)
    return _tc3(agg2, hs2, dinv, b2.reshape(1, D_OUT))
